# baseline (speedup 1.0000x reference)
import functools

import jax
import jax.numpy as jnp
from jax import lax
from jax.experimental import pallas as pl
from jax.experimental.pallas import tpu as pltpu
from jax.experimental.pallas import tpu_sc as plsc

TOPK_FRAC = 0.2
_SUBR = 128
_NC, _NS, _LN = 2, 16, 16
_NW = _NC * _NS
_HR, _HCOL = 256, 128


def _loss_kernel(logits_ref, labels_ref, keys_ref):
    x = logits_ref[0]
    lab = labels_ref[0]
    m = jnp.max(x, axis=0)
    s = jnp.sum(jnp.exp(x - m[None]), axis=0)
    lse = jnp.log(s) + m
    cls = lax.broadcasted_iota(jnp.int32, x.shape, 0)
    picked = jnp.sum(jnp.where(cls == lab[None], x, 0.0), axis=0)
    loss = lse - picked
    keys_ref[0] = lax.bitcast_convert_type(loss, jnp.int32)


def _sc_hist_body(rows_per_w, keys_hbm, cnt_out, sum_out, buf, cnt, hsum,
                  idx_lo, idx_hi, sh_cnt, sh_sum):
    c = lax.axis_index("c")
    s = lax.axis_index("s")
    wid = c * _NS + s
    batch = wid >> 2
    quarter = wid & 3
    iota = lax.broadcasted_iota(jnp.int32, (_LN,), 0)
    zi = jnp.zeros((_LN,), jnp.int32)
    zf = jnp.zeros((_LN,), jnp.float32)

    @plsc.parallel_loop(0, _HR, unroll=4)
    def _(i):
        for u in range(8):
            cnt[i, pl.ds(u * _LN, _LN)] = zi
            hsum[i, pl.ds(u * _LN, _LN)] = zf

    @pl.when(s == 0)
    def _():
        pltpu.sync_copy(cnt, sh_cnt)
        pltpu.sync_copy(hsum, sh_sum)

    plsc.subcore_barrier()

    ones = jnp.ones((_LN,), jnp.int32)
    half = rows_per_w // 2
    for h in range(2):
        pltpu.sync_copy(
            keys_hbm.at[batch, pl.ds(quarter * rows_per_w + h * half, half),
                        :], buf)

        @plsc.parallel_loop(0, half * 512 // _LN, unroll=8)
        def _(i):
            r = lax.shift_right_logical(i, 5)
            u = i & 31
            kv = buf[r, pl.ds(u * _LN, _LN)]
            bkt = lax.shift_right_logical(kv, 16)
            brow = lax.shift_right_logical(bkt, 7)
            bcol = bkt & 127
            plsc.addupdate_scatter(cnt, [brow, bcol], ones)
            plsc.addupdate_scatter(hsum, [brow, bcol],
                                   plsc.bitcast(kv, jnp.float32))

    for u in range(8):
        idx_lo[pl.ds(u * _LN, _LN)] = iota + u * _LN
        idx_hi[pl.ds(u * _LN, _LN)] = iota + 128 + u * _LN

    pltpu.sync_copy(cnt.at[pl.ds(0, 128)], sh_cnt.at[idx_lo], add=True)
    pltpu.sync_copy(cnt.at[pl.ds(128, 128)], sh_cnt.at[idx_hi], add=True)
    pltpu.sync_copy(hsum.at[pl.ds(0, 128)], sh_sum.at[idx_lo], add=True)
    pltpu.sync_copy(hsum.at[pl.ds(128, 128)], sh_sum.at[idx_hi], add=True)
    plsc.subcore_barrier()

    @pl.when(s == 0)
    def _():
        pltpu.sync_copy(sh_cnt, cnt_out.at[c])
        pltpu.sync_copy(sh_sum, sum_out.at[c])


def _finish_kernel(k, cnt_ref, sum_ref, out_ref):
    kf = jnp.float32(k)
    g = (cnt_ref[0] + cnt_ref[1]).astype(jnp.float32)
    gs = sum_ref[0] + sum_ref[1]

    ones_col = jnp.ones((_HCOL, 1), jnp.float32)
    i0 = lax.broadcasted_iota(jnp.int32, (_HR, _HR), 0)
    i1 = lax.broadcasted_iota(jnp.int32, (_HR, _HR), 1)
    mge_r = (i1 >= i0).astype(jnp.float32)
    c0 = lax.broadcasted_iota(jnp.int32, (_HCOL, _HCOL), 0)
    c1 = lax.broadcasted_iota(jnp.int32, (_HCOL, _HCOL), 1)
    mge_c = (c0 >= c1).astype(jnp.float32)

    dot = functools.partial(jax.lax.dot_general,
                            dimension_numbers=(((1,), (0,)), ((), ())),
                            precision=jax.lax.Precision.HIGHEST,
                            preferred_element_type=jnp.float32)

    rt = dot(g, ones_col)
    rf = dot(gs, ones_col)
    s_row = dot(mge_r, rt)
    sf_row = dot(mge_r, rf)
    iota_r = lax.broadcasted_iota(jnp.int32, (_HR, 1), 0)
    rmask = s_row >= kf
    rowsel = jnp.max(jnp.where(rmask, iota_r, -1))

    def _at_r(v):
        return jnp.sum(jnp.where(iota_r == rowsel, v, 0.0))

    cum_at = _at_r(s_row) - _at_r(rt)
    cumf_at = _at_r(sf_row) - _at_r(rf)

    sel2 = lax.broadcasted_iota(jnp.int32, (_HR, _HCOL), 0) == rowsel
    rowc = jnp.sum(jnp.where(sel2, g, 0.0), axis=0, keepdims=True)
    rowf = jnp.sum(jnp.where(sel2, gs, 0.0), axis=0, keepdims=True)
    sc = cum_at + dot(rowc, mge_c)
    scf = dot(rowf, mge_c)
    iota_c = lax.broadcasted_iota(jnp.int32, (1, _HCOL), 1)
    cmask = sc >= kf
    j = jnp.max(jnp.where(cmask, iota_c, -1))

    def _at_c(v):
        return jnp.sum(jnp.where(iota_c == j, v, 0.0))

    c_above = _at_c(sc) - _at_c(rowc)
    s_above = cumf_at + _at_c(scf) - _at_c(rowf)
    b = rowsel * _HCOL + j

    tval = lax.bitcast_convert_type(
        jnp.full((1, 1), (b << 16) | 0x8000, jnp.int32), jnp.float32)
    out_ref[...] = (s_above + (kf - c_above) * tval) * (1.0 / k)


@jax.jit
def kernel(logits, labels):
    b, c, h, w = logits.shape
    total = b * h * w
    k = int(TOPK_FRAC * total)
    nblk = h // _SUBR
    rows_per_w = (b * h) // _NW

    keys = pl.pallas_call(
        _loss_kernel,
        grid=(b, nblk),
        in_specs=[
            pl.BlockSpec((1, c, _SUBR, w), lambda i, j: (i, 0, j, 0)),
            pl.BlockSpec((1, _SUBR, w), lambda i, j: (i, j, 0)),
        ],
        out_specs=pl.BlockSpec((1, _SUBR, w), lambda i, j: (i, j, 0)),
        out_shape=jax.ShapeDtypeStruct((b, h, w), jnp.int32),
        compiler_params=pltpu.CompilerParams(
            dimension_semantics=("arbitrary", "arbitrary")),
    )(logits, labels)

    mesh = plsc.VectorSubcoreMesh(core_axis_name="c", subcore_axis_name="s")
    sc_params = pltpu.CompilerParams(needs_layout_passes=False)

    cnt1, sum1 = pl.kernel(
        functools.partial(_sc_hist_body, rows_per_w),
        out_type=[jax.ShapeDtypeStruct((_NC, _HR, _HCOL), jnp.int32),
                  jax.ShapeDtypeStruct((_NC, _HR, _HCOL), jnp.float32)],
        mesh=mesh,
        scratch_types=[
            pltpu.VMEM((rows_per_w // 2, w), jnp.int32),
            pltpu.VMEM((_HR, _HCOL), jnp.int32),
            pltpu.VMEM((_HR, _HCOL), jnp.float32),
            pltpu.VMEM((128,), jnp.int32),
            pltpu.VMEM((128,), jnp.int32),
            pltpu.VMEM_SHARED((_HR, _HCOL), jnp.int32),
            pltpu.VMEM_SHARED((_HR, _HCOL), jnp.float32),
        ],
        compiler_params=sc_params,
    )(keys)

    out = pl.pallas_call(
        functools.partial(_finish_kernel, k),
        out_shape=jax.ShapeDtypeStruct((1, 1), jnp.float32),
    )(cnt1, sum1)
    return out[0, 0]
